# baseline (speedup 1.0000x reference)
import jax
import jax.numpy as jnp
from jax import lax
from jax.experimental import pallas as pl
from jax.experimental.pallas import tpu as pltpu
from jax.experimental.pallas import tpu_sc as plsc

K = 32
L = 16
W = 128
RPW = W // K
NC = 2
NS = 16
NW = NC * NS
B = 16384
BPW = B // NW
HALF = BPW // 2
CH = 128
NCH = HALF // CH


def _permute(x, idx):
    dnums = lax.GatherDimensionNumbers(
        offset_dims=(), collapsed_slice_dims=(0,), start_index_map=(0,))
    return lax.gather(x, idx[:, None], dnums, (1,),
                      mode=lax.GatherScatterMode.PROMISE_IN_BOUNDS)


def _mf_body(o_hbm, m_hbm, w_hbm, u_hbm, mt_hbm, e_hbm, out_hbm,
             o_idx, m_idx, o4_idx, m4_idx, ooff, moff,
             u_raw, m_raw, e_raw, w_v, out_v, sem):
    wid = lax.axis_index("s") * NC + lax.axis_index("c")
    base = wid * BPW
    lanes = lax.iota(jnp.int32, L)

    for h in range(2):
        hbase = base + h * HALF

        for c in range(NCH):
            pltpu.sync_copy(o_hbm.at[pl.ds(hbase + c * CH, CH)], o_idx.at[c])
            pltpu.sync_copy(m_hbm.at[pl.ds(hbase + c * CH, CH)], m_idx.at[c])
        for c in range(NCH):
            for t in range(CH // L):
                sl = pl.ds(t * L, L)
                fl = pl.ds(c * CH + t * L, L)
                ov = o_idx[c, sl]
                mv = m_idx[c, sl]
                o4_idx[c, sl] = lax.shift_right_logical(ov, 2)
                m4_idx[c, sl] = lax.shift_right_logical(mv, 2)
                ooff[fl] = jnp.bitwise_and(ov, RPW - 1) * K
                moff[fl] = jnp.bitwise_and(mv, RPW - 1) * K

        copies = []
        for c in range(NCH):
            sl = pl.ds(c * CH, CH)
            copies.append(pltpu.async_copy(u_hbm.at[o4_idx.at[c]], u_raw.at[sl], sem))
            copies.append(pltpu.async_copy(e_hbm.at[o4_idx.at[c]], e_raw.at[sl], sem))
            copies.append(pltpu.async_copy(mt_hbm.at[m4_idx.at[c]], m_raw.at[sl], sem))
        pltpu.sync_copy(w_hbm.at[pl.ds(hbase, HALF)], w_v)
        for cp in copies:
            cp.wait()

        def group(g, carry):
            acc_u = jnp.zeros((L,), jnp.float32)
            acc_e = jnp.zeros((L,), jnp.float32)
            qov = ooff[pl.ds(g * L, L)]
            qmv = moff[pl.ds(g * L, L)]
            for j in range(L):
                b = g * L + j
                qo = qov[j]
                qm = qmv[j]
                mva = m_raw[b, pl.ds(qm, L)]
                mvb = m_raw[b, pl.ds(qm + L, L)]
                ua = u_raw[b, pl.ds(qo, L)]
                ub = u_raw[b, pl.ds(qo + L, L)]
                ea = e_raw[b, pl.ds(qo, L)]
                eb = e_raw[b, pl.ds(qo + L, L)]
                pu = mva * ua + mvb * ub
                pe = mva * ea + mvb * eb
                for sh in (8, 4, 2, 1):
                    perm = jnp.bitwise_xor(lanes, sh)
                    pu = pu + _permute(pu, perm)
                    pe = pe + _permute(pe, perm)
                acc_u = jnp.where(lanes == j, pu, acc_u)
                acc_e = jnp.where(lanes == j, pe, acc_e)
            wv = w_v[pl.ds(g * L, L)]
            out_v[pl.ds(g * L, L)] = acc_u * wv + acc_e * (1.0 - wv)
            return carry

        lax.fori_loop(0, HALF // L, group, 0)
        pltpu.sync_copy(out_v, out_hbm.at[pl.ds(hbase, HALF)])


def kernel(o, m, is_user, U, M, E):
    w = is_user.reshape(-1).astype(jnp.float32)
    o32 = o.astype(jnp.int32)
    m32 = m.astype(jnp.int32)
    u2 = U.reshape(-1, W)
    m2 = M.reshape(-1, W)
    e2 = E.reshape(-1, W)
    mesh = plsc.VectorSubcoreMesh(core_axis_name="c", subcore_axis_name="s")
    run = pl.kernel(
        _mf_body,
        mesh=mesh,
        out_type=jax.ShapeDtypeStruct((B,), jnp.float32),
        scratch_types=[
            pltpu.VMEM((NCH, CH), jnp.int32),
            pltpu.VMEM((NCH, CH), jnp.int32),
            pltpu.VMEM((NCH, CH), jnp.int32),
            pltpu.VMEM((NCH, CH), jnp.int32),
            pltpu.VMEM((HALF,), jnp.int32),
            pltpu.VMEM((HALF,), jnp.int32),
            pltpu.VMEM((HALF, W), jnp.float32),
            pltpu.VMEM((HALF, W), jnp.float32),
            pltpu.VMEM((HALF, W), jnp.float32),
            pltpu.VMEM((HALF,), jnp.float32),
            pltpu.VMEM((HALF,), jnp.float32),
            pltpu.SemaphoreType.DMA,
        ],
    )
    return run(o32, m32, w, u2, m2, e2)


# speedup vs baseline: 2.5382x; 2.5382x over previous
import jax
import jax.numpy as jnp
from jax import lax
from jax.experimental import pallas as pl
from jax.experimental.pallas import tpu as pltpu
from jax.experimental.pallas import tpu_sc as plsc

K = 32
L = 16
NC = 2
NS = 16
NW = NC * NS
B = 16384
BPW = B // NW
NU = 1_000_000
NM = 100_000
WIN = 512
UFULL = NU // WIN
MFULL = NM // WIN
UTAIL_ID = UFULL
MTAIL_ID = MFULL
UT_ROWS = NU - UFULL * WIN
MT_ROWS = NM - MFULL * WIN
LCAP = 768

_LANES = None


def _permute(x, idx):
    dnums = lax.GatherDimensionNumbers(
        offset_dims=(), collapsed_slice_dims=(0,), start_index_map=(0,))
    return lax.gather(x, idx[:, None], dnums, (1,),
                      mode=lax.GatherScatterMode.PROMISE_IN_BOUNDS)


def _splat(x):
    return jnp.full((L,), x, jnp.int32)


def _extract(v, l):
    return _permute(v, _splat(l))[0]


def _gather_body(o_hbm, m_hbm, w_hbm, ut_hbm, mt_hbm, et_hbm,
                 utail_hbm, etail_hbm, mtail_hbm,
                 blend_hbm, mrow_hbm,
                 o_full, m_full, w_full, u_win, e_win, ulist, mlist, hrows,
                 sem_s, sem_h):
    wid = lax.axis_index("s") * NC + lax.axis_index("c")
    lanes = lax.iota(jnp.int32, L)
    kb_lo = lax.shift_right_logical(lanes, 3)
    ki_lo = jnp.bitwise_and(lanes, 7)
    kb_hi = kb_lo + 2
    ki_hi = ki_lo

    pltpu.sync_copy(o_hbm, o_full)
    pltpu.sync_copy(m_hbm, m_full)
    pltpu.sync_copy(w_hbm, w_full)

    def make_scan(src_full, dst_list):
        def scan(g, cnt):
            v = src_full[pl.ds(g * L, L)]
            own = jnp.bitwise_and(lax.shift_right_logical(v, 9), NW - 1)
            msk = own == wid

            def hit(h, st):
                c, mc = st
                l = plsc.all_reduce_ffs(mc)[0]
                b = g * L + l
                base16 = jnp.minimum(
                    lax.shift_left(lax.shift_right_logical(c, 4), 4), LCAP - L)
                chunk = dst_list[pl.ds(base16, L)]
                dst_list[pl.ds(base16, L)] = jnp.where(
                    lanes == jnp.bitwise_and(c, L - 1), b, chunk)
                return c + 1, mc & (lanes != l)

            pc = plsc.all_reduce_population_count(msk)[0]
            cnt, _ = lax.fori_loop(0, pc, hit, (cnt, msk))
            return cnt
        return scan

    cnt_u = lax.fori_loop(0, B // L, make_scan(o_full, ulist), 0)
    cnt_m = lax.fori_loop(0, B // L, make_scan(m_full, mlist), 0)
    cnt_u = jnp.minimum(cnt_u, LCAP)
    cnt_m = jnp.minimum(cnt_m, LCAP)

    def serve(win_id, win_ok, hcnt, src_full, src_list, src_cnt, blended,
              dst_hbm):
        ngrp = lax.shift_right_logical(src_cnt + L - 1, 4)

        def grp(i, hc):
            bvec = jnp.bitwise_and(src_list[pl.ds(i * L, L)], B - 1)
            vvec = plsc.load_gather(src_full, [bvec])
            valid = (i * L + lanes) < src_cnt
            msk = (lax.shift_right_logical(vvec, 9) == win_id) & valid & win_ok

            def hit(h, st):
                c, mc = st
                l = plsc.all_reduce_ffs(mc)[0]
                b = _extract(bvec, l)
                col = jnp.bitwise_and(_extract(vvec, l), WIN - 1)
                colv = _splat(col)
                a_lo = plsc.load_gather(u_win, [kb_lo, ki_lo, colv])
                a_hi = plsc.load_gather(u_win, [kb_hi, ki_hi, colv])
                if blended:
                    e_lo = plsc.load_gather(e_win, [kb_lo, ki_lo, colv])
                    e_hi = plsc.load_gather(e_win, [kb_hi, ki_hi, colv])
                    wb = plsc.load_gather(w_full, [_splat(b)])
                    r_lo = a_lo * wb + e_lo * (1.0 - wb)
                    r_hi = a_hi * wb + e_hi * (1.0 - wb)
                else:
                    r_lo, r_hi = a_lo, a_hi
                slot = jnp.minimum(c, LCAP - 1) * K
                hrows[pl.ds(slot, L)] = r_lo
                hrows[pl.ds(slot + L, L)] = r_hi
                pltpu.async_copy(hrows.at[pl.ds(slot, K)],
                                 dst_hbm.at[pl.ds(b * K, K)], sem_h)
                return c + 1, mc & (lanes != l)

            pc = plsc.all_reduce_population_count(msk)[0]
            hc, _ = lax.fori_loop(0, pc, hit, (hc, msk))
            return hc

        return lax.fori_loop(0, ngrp, grp, hcnt)

    def drain(n):
        def one(i, c):
            pltpu.make_async_copy(
                blend_hbm.at[pl.ds(0, K)], hrows.at[pl.ds(0, K)], sem_h).wait()
            return c
        lax.fori_loop(0, n, one, 0)

    def u_window(wi, hcnt):
        win_id = wid + wi * NW
        win_ok = win_id < UFULL

        @pl.when(win_ok)
        def _():
            off = win_id * WIN
            cps = []
            for kb in range(4):
                cps.append(pltpu.async_copy(
                    ut_hbm.at[pl.ds(kb * 8, 8), pl.ds(off, WIN)],
                    u_win.at[kb], sem_s))
                cps.append(pltpu.async_copy(
                    et_hbm.at[pl.ds(kb * 8, 8), pl.ds(off, WIN)],
                    e_win.at[kb], sem_s))
            for cp in cps:
                cp.wait()

        return serve(win_id, win_ok, hcnt, o_full, ulist, cnt_u, True,
                     blend_hbm)

    hcnt = lax.fori_loop(0, UFULL // NW + 1, u_window, 0)

    @pl.when(wid == UTAIL_ID % NW)
    def _():
        cps = []
        for kb in range(4):
            cps.append(pltpu.async_copy(
                utail_hbm.at[pl.ds(kb * 8, 8), :],
                u_win.at[kb, :, pl.ds(0, 128)], sem_s))
            cps.append(pltpu.async_copy(
                etail_hbm.at[pl.ds(kb * 8, 8), :],
                e_win.at[kb, :, pl.ds(0, 128)], sem_s))
        for cp in cps:
            cp.wait()

    hcnt = serve(UTAIL_ID, wid == UTAIL_ID % NW, hcnt, o_full, ulist, cnt_u,
                 True, blend_hbm)
    drain(hcnt)

    def m_window(wi, hcnt):
        win_id = wid + wi * NW
        win_ok = win_id < MFULL

        @pl.when(win_ok)
        def _():
            off = win_id * WIN
            cps = [pltpu.async_copy(
                mt_hbm.at[pl.ds(kb * 8, 8), pl.ds(off, WIN)],
                u_win.at[kb], sem_s) for kb in range(4)]
            for cp in cps:
                cp.wait()

        return serve(win_id, win_ok, hcnt, m_full, mlist, cnt_m, False,
                     mrow_hbm)

    mcnt = lax.fori_loop(0, MFULL // NW + 1, m_window, 0)

    @pl.when(wid == MTAIL_ID % NW)
    def _():
        cps = [pltpu.async_copy(
            mtail_hbm.at[pl.ds(kb * 8, 8), :],
            u_win.at[kb, :, pl.ds(0, 256)], sem_s) for kb in range(4)]
        for cp in cps:
            cp.wait()

    mcnt = serve(MTAIL_ID, wid == MTAIL_ID % NW, mcnt, m_full, mlist, cnt_m,
                 False, mrow_hbm)
    drain(mcnt)


def _dot_body(blend_hbm, mrow_hbm, out_hbm, b_v, m_v, out_v, sem):
    wid = lax.axis_index("s") * NC + lax.axis_index("c")
    base = wid * BPW
    lanes = lax.iota(jnp.int32, L)
    pltpu.sync_copy(blend_hbm.at[pl.ds(base * K, BPW * K)], b_v)
    pltpu.sync_copy(mrow_hbm.at[pl.ds(base * K, BPW * K)], m_v)

    def group(g, carry):
        acc = jnp.zeros((L,), jnp.float32)
        for j in range(L):
            i = (g * L + j) * K
            blo = b_v[pl.ds(i, L)]
            bhi = b_v[pl.ds(i + L, L)]
            mlo = m_v[pl.ds(i, L)]
            mhi = m_v[pl.ds(i + L, L)]
            s = jnp.sum(blo * mlo + bhi * mhi)
            acc = jnp.where(lanes == j, s, acc)
        out_v[pl.ds(g * L, L)] = acc
        return carry

    lax.fori_loop(0, BPW // L, group, 0)
    pltpu.sync_copy(out_v, out_hbm.at[pl.ds(base, BPW)])


def kernel(o, m, is_user, U, M, E):
    w = is_user.reshape(-1).astype(jnp.float32)
    o32 = o.astype(jnp.int32)
    m32 = m.astype(jnp.int32)
    ut = U.T
    mt = M.T
    et = E.T
    utail = jnp.pad(U[UFULL * WIN:].T, ((0, 0), (0, 128 - UT_ROWS)))
    etail = jnp.pad(E[UFULL * WIN:].T, ((0, 0), (0, 128 - UT_ROWS)))
    mtail = jnp.pad(M[MFULL * WIN:].T, ((0, 0), (0, 256 - MT_ROWS)))
    mesh = plsc.VectorSubcoreMesh(core_axis_name="c", subcore_axis_name="s")
    params = pltpu.CompilerParams(needs_layout_passes=False)

    gather = pl.kernel(
        _gather_body,
        mesh=mesh,
        compiler_params=params,
        out_type=(jax.ShapeDtypeStruct((B * K,), jnp.float32),
                  jax.ShapeDtypeStruct((B * K,), jnp.float32)),
        scratch_types=[
            pltpu.VMEM((B,), jnp.int32),
            pltpu.VMEM((B,), jnp.int32),
            pltpu.VMEM((B,), jnp.float32),
            pltpu.VMEM((4, 8, WIN), jnp.float32),
            pltpu.VMEM((4, 8, WIN), jnp.float32),
            pltpu.VMEM((LCAP,), jnp.int32),
            pltpu.VMEM((LCAP,), jnp.int32),
            pltpu.VMEM((LCAP * K,), jnp.float32),
            pltpu.SemaphoreType.DMA,
            pltpu.SemaphoreType.DMA,
        ],
    )
    blend, mrow = gather(o32, m32, w, ut, mt, et, utail, etail, mtail)

    dot = pl.kernel(
        _dot_body,
        mesh=mesh,
        compiler_params=params,
        out_type=jax.ShapeDtypeStruct((B,), jnp.float32),
        scratch_types=[
            pltpu.VMEM((BPW * K,), jnp.float32),
            pltpu.VMEM((BPW * K,), jnp.float32),
            pltpu.VMEM((BPW,), jnp.float32),
            pltpu.SemaphoreType.DMA,
        ],
    )
    return dot(blend, mrow)


# speedup vs baseline: 3.0148x; 1.1878x over previous
import jax
import jax.numpy as jnp
from jax import lax
from jax.experimental import pallas as pl
from jax.experimental.pallas import tpu as pltpu
from jax.experimental.pallas import tpu_sc as plsc

K = 32
L = 16
NC = 2
NS = 16
NW = NC * NS
B = 16384
BPW = B // NW
NU = 1_000_000
NM = 100_000
WIN = 1024
WSH = 10
UFULL = NU // WIN
MFULL = NM // WIN
UT_ROWS = NU - UFULL * WIN
MT_ROWS = NM - MFULL * WIN
UT_PAD = 640
MT_PAD = 768
LCAP = 768


def _permute(x, idx):
    dnums = lax.GatherDimensionNumbers(
        offset_dims=(), collapsed_slice_dims=(0,), start_index_map=(0,))
    return lax.gather(x, idx[:, None], dnums, (1,),
                      mode=lax.GatherScatterMode.PROMISE_IN_BOUNDS)


def _splat(x):
    return jnp.full((L,), x, jnp.int32)


def _extract(v, l):
    return _permute(v, _splat(l))[0]


def _gather_body(o_hbm, m_hbm, ut_hbm, mt_hbm, et_hbm,
                 utail_hbm, etail_hbm, mtail_hbm,
                 blend_hbm, mrow_hbm,
                 o_full, m_full, u_win, e_win, ulist, mlist, hrows,
                 sem_s, sem_h):
    wid = lax.axis_index("s") * NC + lax.axis_index("c")
    lanes = lax.iota(jnp.int32, L)
    k_lo = lanes
    k_hi = lanes + L

    pltpu.sync_copy(o_hbm, o_full)
    pltpu.sync_copy(m_hbm, m_full)

    def make_scan(src_full, dst_list):
        def scan(g, cnt):
            v = src_full[pl.ds(g * L, L)]
            own = jnp.bitwise_and(lax.shift_right_logical(v, WSH), NW - 1)
            msk = own == wid

            def hit(h, st):
                c, mc = st
                l = plsc.all_reduce_ffs(mc)[0]
                b = g * L + l
                base16 = jnp.minimum(
                    lax.shift_left(lax.shift_right_logical(c, 4), 4), LCAP - L)
                chunk = dst_list[pl.ds(base16, L)]
                dst_list[pl.ds(base16, L)] = jnp.where(
                    lanes == jnp.bitwise_and(c, L - 1), b, chunk)
                return c + 1, mc & (lanes != l)

            pc = plsc.all_reduce_population_count(msk)[0]
            cnt, _ = lax.fori_loop(0, pc, hit, (cnt, msk))
            return cnt
        return scan

    cnt_u = lax.fori_loop(0, B // L, make_scan(o_full, ulist), 0)
    cnt_m = lax.fori_loop(0, B // L, make_scan(m_full, mlist), 0)
    cnt_u = jnp.minimum(cnt_u, LCAP)
    cnt_m = jnp.minimum(cnt_m, LCAP)

    def serve(win_id, win_ok, hcnt, src_full, src_list, src_cnt, blended,
              dst_hbm):
        ngrp = lax.shift_right_logical(src_cnt + L - 1, 4)

        def grp(i, hc):
            bvec = jnp.bitwise_and(src_list[pl.ds(i * L, L)], B - 1)
            vvec = plsc.load_gather(src_full, [bvec])
            win_of = jnp.bitwise_and(
                lax.shift_right_logical(vvec, WSH), 1023)
            valid = (i * L + lanes) < src_cnt
            msk = (win_of == win_id) & valid & win_ok

            def hit(h, st):
                c, mc = st
                l = plsc.all_reduce_ffs(mc)[0]
                b = _extract(bvec, l)
                v = _extract(vvec, l)
                colv = _splat(jnp.bitwise_and(v, WIN - 1))
                a_lo = plsc.load_gather(u_win, [k_lo, colv])
                a_hi = plsc.load_gather(u_win, [k_hi, colv])
                if blended:
                    e_lo = plsc.load_gather(e_win, [k_lo, colv])
                    e_hi = plsc.load_gather(e_win, [k_hi, colv])
                    wf = jnp.float32(
                        jnp.bitwise_and(lax.shift_right_logical(v, 30), 1))
                    wb = jnp.full((L,), wf, jnp.float32)
                    r_lo = a_lo * wb + e_lo * (1.0 - wb)
                    r_hi = a_hi * wb + e_hi * (1.0 - wb)
                else:
                    r_lo, r_hi = a_lo, a_hi
                slot = jnp.minimum(c, LCAP - 1) * K
                hrows[pl.ds(slot, L)] = r_lo
                hrows[pl.ds(slot + L, L)] = r_hi
                pltpu.async_copy(hrows.at[pl.ds(slot, K)],
                                 dst_hbm.at[pl.ds(b * K, K)], sem_h)
                return c + 1, mc & (lanes != l)

            pc = plsc.all_reduce_population_count(msk)[0]
            hc, _ = lax.fori_loop(0, pc, hit, (hc, msk))
            return hc

        return lax.fori_loop(0, ngrp, grp, hcnt)

    def drain(n):
        def one(i, c):
            pltpu.make_async_copy(
                blend_hbm.at[pl.ds(0, K)], hrows.at[pl.ds(0, K)], sem_h).wait()
            return c
        lax.fori_loop(0, n, one, 0)

    def u_window(wi, hcnt):
        win_id = wid + wi * NW
        win_ok = win_id < UFULL

        @pl.when(win_ok)
        def _():
            off = win_id * WIN
            c1 = pltpu.async_copy(ut_hbm.at[:, pl.ds(off, WIN)], u_win, sem_s)
            c2 = pltpu.async_copy(et_hbm.at[:, pl.ds(off, WIN)], e_win, sem_s)
            c1.wait()
            c2.wait()

        return serve(win_id, win_ok, hcnt, o_full, ulist, cnt_u, True,
                     blend_hbm)

    hcnt = lax.fori_loop(0, UFULL // NW + 1, u_window, 0)

    @pl.when(wid == UFULL % NW)
    def _():
        c1 = pltpu.async_copy(utail_hbm, u_win.at[:, pl.ds(0, UT_PAD)], sem_s)
        c2 = pltpu.async_copy(etail_hbm, e_win.at[:, pl.ds(0, UT_PAD)], sem_s)
        c1.wait()
        c2.wait()

    hcnt = serve(UFULL, wid == UFULL % NW, hcnt, o_full, ulist, cnt_u,
                 True, blend_hbm)
    drain(hcnt)

    def m_window(wi, hcnt):
        win_id = wid + wi * NW
        win_ok = win_id < MFULL

        @pl.when(win_ok)
        def _():
            off = win_id * WIN
            pltpu.async_copy(
                mt_hbm.at[:, pl.ds(off, WIN)], u_win, sem_s).wait()

        return serve(win_id, win_ok, hcnt, m_full, mlist, cnt_m, False,
                     mrow_hbm)

    mcnt = lax.fori_loop(0, MFULL // NW + 1, m_window, 0)

    @pl.when(wid == MFULL % NW)
    def _():
        pltpu.async_copy(mtail_hbm, u_win.at[:, pl.ds(0, MT_PAD)], sem_s).wait()

    mcnt = serve(MFULL, wid == MFULL % NW, mcnt, m_full, mlist, cnt_m,
                 False, mrow_hbm)
    drain(mcnt)


def _dot_body(blend_hbm, mrow_hbm, out_hbm, b_v, m_v, out_v, sem):
    wid = lax.axis_index("s") * NC + lax.axis_index("c")
    base = wid * BPW
    lanes = lax.iota(jnp.int32, L)
    pltpu.sync_copy(blend_hbm.at[pl.ds(base * K, BPW * K)], b_v)
    pltpu.sync_copy(mrow_hbm.at[pl.ds(base * K, BPW * K)], m_v)

    def group(g, carry):
        acc = jnp.zeros((L,), jnp.float32)
        for j in range(L):
            i = (g * L + j) * K
            blo = b_v[pl.ds(i, L)]
            bhi = b_v[pl.ds(i + L, L)]
            mlo = m_v[pl.ds(i, L)]
            mhi = m_v[pl.ds(i + L, L)]
            s = jnp.sum(blo * mlo + bhi * mhi)
            acc = jnp.where(lanes == j, s, acc)
        out_v[pl.ds(g * L, L)] = acc
        return carry

    lax.fori_loop(0, BPW // L, group, 0)
    pltpu.sync_copy(out_v, out_hbm.at[pl.ds(base, BPW)])


def kernel(o, m, is_user, U, M, E):
    wbit = is_user.reshape(-1).astype(jnp.int32)
    opk = o.astype(jnp.int32) | lax.shift_left(wbit, 30)
    m32 = m.astype(jnp.int32)
    ut = U.T
    mt = M.T
    et = E.T
    utail = jnp.pad(U[UFULL * WIN:].T, ((0, 0), (0, UT_PAD - UT_ROWS)))
    etail = jnp.pad(E[UFULL * WIN:].T, ((0, 0), (0, UT_PAD - UT_ROWS)))
    mtail = jnp.pad(M[MFULL * WIN:].T, ((0, 0), (0, MT_PAD - MT_ROWS)))
    mesh = plsc.VectorSubcoreMesh(core_axis_name="c", subcore_axis_name="s")
    params = pltpu.CompilerParams(needs_layout_passes=False)

    gather = pl.kernel(
        _gather_body,
        mesh=mesh,
        compiler_params=params,
        out_type=(jax.ShapeDtypeStruct((B * K,), jnp.float32),
                  jax.ShapeDtypeStruct((B * K,), jnp.float32)),
        scratch_types=[
            pltpu.VMEM((B,), jnp.int32),
            pltpu.VMEM((B,), jnp.int32),
            pltpu.VMEM((K, WIN), jnp.float32),
            pltpu.VMEM((K, WIN), jnp.float32),
            pltpu.VMEM((LCAP,), jnp.int32),
            pltpu.VMEM((LCAP,), jnp.int32),
            pltpu.VMEM((LCAP * K,), jnp.float32),
            pltpu.SemaphoreType.DMA,
            pltpu.SemaphoreType.DMA,
        ],
    )
    blend, mrow = gather(opk, m32, ut, mt, et, utail, etail, mtail)

    dot = pl.kernel(
        _dot_body,
        mesh=mesh,
        compiler_params=params,
        out_type=jax.ShapeDtypeStruct((B,), jnp.float32),
        scratch_types=[
            pltpu.VMEM((BPW * K,), jnp.float32),
            pltpu.VMEM((BPW * K,), jnp.float32),
            pltpu.VMEM((BPW,), jnp.float32),
            pltpu.SemaphoreType.DMA,
        ],
    )
    return dot(blend, mrow)


# speedup vs baseline: 4.2062x; 1.3952x over previous
import jax
import jax.numpy as jnp
from jax import lax
from jax.experimental import pallas as pl
from jax.experimental.pallas import tpu as pltpu
from jax.experimental.pallas import tpu_sc as plsc

K = 32
L = 16
NC = 2
NS = 16
NW = NC * NS
B = 16384
BPW = B // NW
NU = 1_000_000
NM = 100_000
WIN = 512
WSH = 9
UFULL = NU // WIN
MFULL = NM // WIN
UT_ROWS = NU - UFULL * WIN
MT_ROWS = NM - MFULL * WIN
UT_PAD = 128
MT_PAD = 256
UPAIRS = (UFULL // NW + 2) // 2
MPAIRS = (MFULL // NW + 2) // 2
LCAP = 768


def _permute(x, idx):
    dnums = lax.GatherDimensionNumbers(
        offset_dims=(), collapsed_slice_dims=(0,), start_index_map=(0,))
    return lax.gather(x, idx[:, None], dnums, (1,),
                      mode=lax.GatherScatterMode.PROMISE_IN_BOUNDS)


def _splat(x):
    return jnp.full((L,), x, jnp.int32)


def _extract(v, l):
    return _permute(v, _splat(l))[0]


def _gather_body(o_hbm, m_hbm, ut_hbm, mt_hbm, et_hbm,
                 utail_hbm, etail_hbm, mtail_hbm,
                 blend_hbm, mrow_hbm,
                 o_full, m_full, u_win0, u_win1, e_win0, e_win1,
                 ulist, mlist, hrows, sem0, sem1, sem_h):
    wid = lax.axis_index("s") * NC + lax.axis_index("c")
    lanes = lax.iota(jnp.int32, L)
    k_lo = lanes
    k_hi = lanes + L
    u_bufs = (u_win0, u_win1)
    e_bufs = (e_win0, e_win1)
    sems = (sem0, sem1)

    pltpu.sync_copy(o_hbm, o_full)
    pltpu.sync_copy(m_hbm, m_full)

    def make_scan(src_full, dst_list):
        def scan(g, cnt):
            v = src_full[pl.ds(g * L, L)]
            own = jnp.bitwise_and(lax.shift_right_logical(v, WSH), NW - 1)
            msk = own == wid
            pos = cnt + plsc.cumsum(msk.astype(jnp.int32)) - 1
            pos = jnp.minimum(pos, LCAP - 1)
            plsc.store_scatter(dst_list, [pos], g * L + lanes, mask=msk)
            return cnt + plsc.all_reduce_population_count(msk)[0]
        return scan

    cnt_u = lax.fori_loop(0, B // L, make_scan(o_full, ulist), 0)
    cnt_m = lax.fori_loop(0, B // L, make_scan(m_full, mlist), 0)
    cnt_u = jnp.minimum(cnt_u, LCAP)
    cnt_m = jnp.minimum(cnt_m, LCAP)

    def serve(win_id, win_ok, hcnt, src_full, src_list, src_cnt, blended,
              dst_hbm, u_ref, e_ref):
        ngrp = lax.shift_right_logical(src_cnt + L - 1, 4)

        def grp(i, hc):
            bvec = jnp.bitwise_and(src_list[pl.ds(i * L, L)], B - 1)
            vvec = plsc.load_gather(src_full, [bvec])
            win_of = jnp.bitwise_and(
                lax.shift_right_logical(vvec, WSH), 2047)
            valid = (i * L + lanes) < src_cnt
            msk = (win_of == win_id) & valid & win_ok

            def hit(h, st):
                c, mc = st
                l = plsc.all_reduce_ffs(mc)[0]
                b = _extract(bvec, l)
                v = _extract(vvec, l)
                colv = _splat(jnp.bitwise_and(v, WIN - 1))
                a_lo = plsc.load_gather(u_ref, [k_lo, colv])
                a_hi = plsc.load_gather(u_ref, [k_hi, colv])
                if blended:
                    e_lo = plsc.load_gather(e_ref, [k_lo, colv])
                    e_hi = plsc.load_gather(e_ref, [k_hi, colv])
                    wf = jnp.float32(
                        jnp.bitwise_and(lax.shift_right_logical(v, 30), 1))
                    wb = jnp.full((L,), wf, jnp.float32)
                    r_lo = a_lo * wb + e_lo * (1.0 - wb)
                    r_hi = a_hi * wb + e_hi * (1.0 - wb)
                else:
                    r_lo, r_hi = a_lo, a_hi
                slot = jnp.minimum(c, LCAP - 1) * K
                hrows[pl.ds(slot, L)] = r_lo
                hrows[pl.ds(slot + L, L)] = r_hi
                pltpu.async_copy(hrows.at[pl.ds(slot, K)],
                                 dst_hbm.at[pl.ds(b * K, K)], sem_h)
                return c + 1, mc & (lanes != l)

            pc = plsc.all_reduce_population_count(msk)[0]
            hc, _ = lax.fori_loop(0, pc, hit, (hc, msk))
            return hc

        return lax.fori_loop(0, ngrp, grp, hcnt)

    def drain(n):
        def one(i, c):
            pltpu.make_async_copy(
                blend_hbm.at[pl.ds(0, K)], hrows.at[pl.ds(0, K)], sem_h).wait()
            return c
        lax.fori_loop(0, n, one, 0)

    def u_fire(win_id, p):
        @pl.when(win_id < UFULL)
        def _():
            off = win_id * WIN
            pltpu.async_copy(ut_hbm.at[:, pl.ds(off, WIN)], u_bufs[p],
                             sems[p])
            pltpu.async_copy(et_hbm.at[:, pl.ds(off, WIN)], e_bufs[p],
                             sems[p])

    def u_wait(win_id, p):
        @pl.when(win_id < UFULL)
        def _():
            pltpu.make_async_copy(
                ut_hbm.at[:, pl.ds(0, WIN)], u_bufs[p], sems[p]).wait()
            pltpu.make_async_copy(
                et_hbm.at[:, pl.ds(0, WIN)], e_bufs[p], sems[p]).wait()

    u_fire(wid, 0)

    def u_pair(wi, hcnt):
        id0 = wid + (2 * wi) * NW
        id1 = wid + (2 * wi + 1) * NW
        id2 = wid + (2 * wi + 2) * NW
        u_fire(id1, 1)
        u_wait(id0, 0)
        hcnt = serve(id0, id0 < UFULL, hcnt, o_full, ulist, cnt_u, True,
                     blend_hbm, u_win0, e_win0)
        u_fire(id2, 0)
        u_wait(id1, 1)
        hcnt = serve(id1, id1 < UFULL, hcnt, o_full, ulist, cnt_u, True,
                     blend_hbm, u_win1, e_win1)
        return hcnt

    hcnt = lax.fori_loop(0, UPAIRS, u_pair, 0)

    @pl.when(wid == UFULL % NW)
    def _():
        c1 = pltpu.async_copy(utail_hbm, u_win0.at[:, pl.ds(0, UT_PAD)], sem0)
        c2 = pltpu.async_copy(etail_hbm, e_win0.at[:, pl.ds(0, UT_PAD)], sem0)
        c1.wait()
        c2.wait()

    hcnt = serve(UFULL, wid == UFULL % NW, hcnt, o_full, ulist, cnt_u,
                 True, blend_hbm, u_win0, e_win0)
    drain(hcnt)

    def m_fire(win_id, p):
        @pl.when(win_id < MFULL)
        def _():
            pltpu.async_copy(mt_hbm.at[:, pl.ds(win_id * WIN, WIN)],
                             u_bufs[p], sems[p])

    def m_wait(win_id, p):
        @pl.when(win_id < MFULL)
        def _():
            pltpu.make_async_copy(
                mt_hbm.at[:, pl.ds(0, WIN)], u_bufs[p], sems[p]).wait()

    m_fire(wid, 0)

    def m_pair(wi, mcnt):
        id0 = wid + (2 * wi) * NW
        id1 = wid + (2 * wi + 1) * NW
        id2 = wid + (2 * wi + 2) * NW
        m_fire(id1, 1)
        m_wait(id0, 0)
        mcnt = serve(id0, id0 < MFULL, mcnt, m_full, mlist, cnt_m, False,
                     mrow_hbm, u_win0, e_win0)
        m_fire(id2, 0)
        m_wait(id1, 1)
        mcnt = serve(id1, id1 < MFULL, mcnt, m_full, mlist, cnt_m, False,
                     mrow_hbm, u_win1, e_win1)
        return mcnt

    mcnt = lax.fori_loop(0, MPAIRS, m_pair, 0)

    @pl.when(wid == MFULL % NW)
    def _():
        pltpu.async_copy(mtail_hbm, u_win0.at[:, pl.ds(0, MT_PAD)],
                         sem0).wait()

    mcnt = serve(MFULL, wid == MFULL % NW, mcnt, m_full, mlist, cnt_m,
                 False, mrow_hbm, u_win0, e_win0)
    drain(mcnt)


def _dot_body(blend_hbm, mrow_hbm, out_hbm, b_v, m_v, out_v, sem):
    wid = lax.axis_index("s") * NC + lax.axis_index("c")
    base = wid * BPW
    lanes = lax.iota(jnp.int32, L)
    pltpu.sync_copy(blend_hbm.at[pl.ds(base * K, BPW * K)], b_v)
    pltpu.sync_copy(mrow_hbm.at[pl.ds(base * K, BPW * K)], m_v)

    def group(g, carry):
        acc = jnp.zeros((L,), jnp.float32)
        for j in range(L):
            i = (g * L + j) * K
            blo = b_v[pl.ds(i, L)]
            bhi = b_v[pl.ds(i + L, L)]
            mlo = m_v[pl.ds(i, L)]
            mhi = m_v[pl.ds(i + L, L)]
            s = jnp.sum(blo * mlo + bhi * mhi)
            acc = jnp.where(lanes == j, s, acc)
        out_v[pl.ds(g * L, L)] = acc
        return carry

    lax.fori_loop(0, BPW // L, group, 0)
    pltpu.sync_copy(out_v, out_hbm.at[pl.ds(base, BPW)])


def kernel(o, m, is_user, U, M, E):
    wbit = is_user.reshape(-1).astype(jnp.int32)
    opk = o.astype(jnp.int32) | lax.shift_left(wbit, 30)
    m32 = m.astype(jnp.int32)
    ut = U.T
    mt = M.T
    et = E.T
    utail = jnp.pad(U[UFULL * WIN:].T, ((0, 0), (0, UT_PAD - UT_ROWS)))
    etail = jnp.pad(E[UFULL * WIN:].T, ((0, 0), (0, UT_PAD - UT_ROWS)))
    mtail = jnp.pad(M[MFULL * WIN:].T, ((0, 0), (0, MT_PAD - MT_ROWS)))
    mesh = plsc.VectorSubcoreMesh(core_axis_name="c", subcore_axis_name="s")
    params = pltpu.CompilerParams(needs_layout_passes=False)

    gather = pl.kernel(
        _gather_body,
        mesh=mesh,
        compiler_params=params,
        out_type=(jax.ShapeDtypeStruct((B * K,), jnp.float32),
                  jax.ShapeDtypeStruct((B * K,), jnp.float32)),
        scratch_types=[
            pltpu.VMEM((B,), jnp.int32),
            pltpu.VMEM((B,), jnp.int32),
            pltpu.VMEM((K, WIN), jnp.float32),
            pltpu.VMEM((K, WIN), jnp.float32),
            pltpu.VMEM((K, WIN), jnp.float32),
            pltpu.VMEM((K, WIN), jnp.float32),
            pltpu.VMEM((LCAP,), jnp.int32),
            pltpu.VMEM((LCAP,), jnp.int32),
            pltpu.VMEM((LCAP * K,), jnp.float32),
            pltpu.SemaphoreType.DMA,
            pltpu.SemaphoreType.DMA,
            pltpu.SemaphoreType.DMA,
        ],
    )
    blend, mrow = gather(opk, m32, ut, mt, et, utail, etail, mtail)

    dot = pl.kernel(
        _dot_body,
        mesh=mesh,
        compiler_params=params,
        out_type=jax.ShapeDtypeStruct((B,), jnp.float32),
        scratch_types=[
            pltpu.VMEM((BPW * K,), jnp.float32),
            pltpu.VMEM((BPW * K,), jnp.float32),
            pltpu.VMEM((BPW,), jnp.float32),
            pltpu.SemaphoreType.DMA,
        ],
    )
    return dot(blend, mrow)


# speedup vs baseline: 4.2438x; 1.0089x over previous
import jax
import jax.numpy as jnp
from jax import lax
from jax.experimental import pallas as pl
from jax.experimental.pallas import tpu as pltpu
from jax.experimental.pallas import tpu_sc as plsc

K = 32
L = 16
NC = 2
NS = 16
NW = NC * NS
B = 16384
BPW = B // NW
NU = 1_000_000
NM = 100_000
WIN = 512
WSH = 9
UFULL = NU // WIN
MFULL = NM // WIN
UT_ROWS = NU - UFULL * WIN
MT_ROWS = NM - MFULL * WIN
UT_PAD = 128
MT_PAD = 256
UPAIRS = (UFULL // NW + 2) // 2
MPAIRS = (MFULL // NW + 2) // 2
LCAP = 768


def _permute(x, idx):
    dnums = lax.GatherDimensionNumbers(
        offset_dims=(), collapsed_slice_dims=(0,), start_index_map=(0,))
    return lax.gather(x, idx[:, None], dnums, (1,),
                      mode=lax.GatherScatterMode.PROMISE_IN_BOUNDS)


def _splat(x):
    return jnp.full((L,), x, jnp.int32)


def _extract(v, l):
    return _permute(v, _splat(l))[0]


def _gather_body(o_hbm, m_hbm, ut_hbm, mt_hbm, et_hbm,
                 utail_hbm, etail_hbm, mtail_hbm,
                 blend_hbm, mrow_hbm,
                 o_full, m_full, u_win0, u_win1, e_win0, e_win1,
                 ulist, mlist, hrows, sem0, sem1, sem_h):
    wid = lax.axis_index("s") * NC + lax.axis_index("c")
    lanes = lax.iota(jnp.int32, L)
    k_lo = lanes
    k_hi = lanes + L
    u_bufs = (u_win0, u_win1)
    e_bufs = (e_win0, e_win1)
    sems = (sem0, sem1)

    pltpu.sync_copy(o_hbm, o_full)
    pltpu.sync_copy(m_hbm, m_full)

    @pl.when(wid < UFULL)
    def _():
        pltpu.async_copy(ut_hbm.at[:, pl.ds(wid * WIN, WIN)], u_win0, sem0)
        pltpu.async_copy(et_hbm.at[:, pl.ds(wid * WIN, WIN)], e_win0, sem0)

    def make_scan(src_full, dst_list):
        def scan(g, cnt):
            v = src_full[pl.ds(g * L, L)]
            own = jnp.bitwise_and(lax.shift_right_logical(v, WSH), NW - 1)
            msk = own == wid
            pos = cnt + plsc.cumsum(msk.astype(jnp.int32)) - 1
            pos = jnp.minimum(pos, LCAP - 1)
            plsc.store_scatter(dst_list, [pos], g * L + lanes, mask=msk)
            return cnt + plsc.all_reduce_population_count(msk)[0]
        return scan

    cnt_u = lax.fori_loop(0, B // L, make_scan(o_full, ulist), 0)
    cnt_m = lax.fori_loop(0, B // L, make_scan(m_full, mlist), 0)
    cnt_u = jnp.minimum(cnt_u, LCAP)
    cnt_m = jnp.minimum(cnt_m, LCAP)

    def serve(win_id, win_ok, hcnt, src_full, src_list, src_cnt, blended,
              dst_hbm, u_ref, e_ref):
        ngrp = lax.shift_right_logical(src_cnt + L - 1, 4)

        def grp(i, hc):
            bvec = jnp.bitwise_and(src_list[pl.ds(i * L, L)], B - 1)
            vvec = plsc.load_gather(src_full, [bvec])
            win_of = jnp.bitwise_and(
                lax.shift_right_logical(vvec, WSH), 2047)
            valid = (i * L + lanes) < src_cnt
            msk = (win_of == win_id) & valid & win_ok

            def hit(h, st):
                c, mc = st
                l = plsc.all_reduce_ffs(mc)[0]
                b = _extract(bvec, l)
                v = _extract(vvec, l)
                colv = _splat(jnp.bitwise_and(v, WIN - 1))
                a_lo = plsc.load_gather(u_ref, [k_lo, colv])
                a_hi = plsc.load_gather(u_ref, [k_hi, colv])
                if blended:
                    e_lo = plsc.load_gather(e_ref, [k_lo, colv])
                    e_hi = plsc.load_gather(e_ref, [k_hi, colv])
                    wf = jnp.float32(
                        jnp.bitwise_and(lax.shift_right_logical(v, 30), 1))
                    wb = jnp.full((L,), wf, jnp.float32)
                    r_lo = a_lo * wb + e_lo * (1.0 - wb)
                    r_hi = a_hi * wb + e_hi * (1.0 - wb)
                else:
                    r_lo, r_hi = a_lo, a_hi
                slot = jnp.minimum(c, LCAP - 1) * K
                hrows[pl.ds(slot, L)] = r_lo
                hrows[pl.ds(slot + L, L)] = r_hi
                pltpu.async_copy(hrows.at[pl.ds(slot, K)],
                                 dst_hbm.at[pl.ds(b * K, K)], sem_h)
                return c + 1, mc & (lanes != l)

            pc = plsc.all_reduce_population_count(msk)[0]
            hc, _ = lax.fori_loop(0, pc, hit, (hc, msk))
            return hc

        return lax.fori_loop(0, ngrp, grp, hcnt)

    def drain(n):
        def one(i, c):
            pltpu.make_async_copy(
                blend_hbm.at[pl.ds(0, K)], hrows.at[pl.ds(0, K)], sem_h).wait()
            return c
        lax.fori_loop(0, n, one, 0)

    def u_fire(win_id, p):
        @pl.when(win_id < UFULL)
        def _():
            off = win_id * WIN
            pltpu.async_copy(ut_hbm.at[:, pl.ds(off, WIN)], u_bufs[p],
                             sems[p])
            pltpu.async_copy(et_hbm.at[:, pl.ds(off, WIN)], e_bufs[p],
                             sems[p])

    def u_wait(win_id, p):
        @pl.when(win_id < UFULL)
        def _():
            pltpu.make_async_copy(
                ut_hbm.at[:, pl.ds(0, WIN)], u_bufs[p], sems[p]).wait()
            pltpu.make_async_copy(
                et_hbm.at[:, pl.ds(0, WIN)], e_bufs[p], sems[p]).wait()

    def u_pair(wi, hcnt):
        id0 = wid + (2 * wi) * NW
        id1 = wid + (2 * wi + 1) * NW
        id2 = wid + (2 * wi + 2) * NW
        u_fire(id1, 1)
        u_wait(id0, 0)
        hcnt = serve(id0, id0 < UFULL, hcnt, o_full, ulist, cnt_u, True,
                     blend_hbm, u_win0, e_win0)
        u_fire(id2, 0)
        u_wait(id1, 1)
        hcnt = serve(id1, id1 < UFULL, hcnt, o_full, ulist, cnt_u, True,
                     blend_hbm, u_win1, e_win1)
        return hcnt

    hcnt = lax.fori_loop(0, UPAIRS, u_pair, 0)

    @pl.when(wid == UFULL % NW)
    def _():
        c1 = pltpu.async_copy(utail_hbm, u_win0.at[:, pl.ds(0, UT_PAD)], sem0)
        c2 = pltpu.async_copy(etail_hbm, e_win0.at[:, pl.ds(0, UT_PAD)], sem0)
        c1.wait()
        c2.wait()

    hcnt = serve(UFULL, wid == UFULL % NW, hcnt, o_full, ulist, cnt_u,
                 True, blend_hbm, u_win0, e_win0)
    drain(hcnt)

    def m_fire(win_id, p):
        @pl.when(win_id < MFULL)
        def _():
            pltpu.async_copy(mt_hbm.at[:, pl.ds(win_id * WIN, WIN)],
                             u_bufs[p], sems[p])

    def m_wait(win_id, p):
        @pl.when(win_id < MFULL)
        def _():
            pltpu.make_async_copy(
                mt_hbm.at[:, pl.ds(0, WIN)], u_bufs[p], sems[p]).wait()

    m_fire(wid, 0)

    def m_pair(wi, mcnt):
        id0 = wid + (2 * wi) * NW
        id1 = wid + (2 * wi + 1) * NW
        id2 = wid + (2 * wi + 2) * NW
        m_fire(id1, 1)
        m_wait(id0, 0)
        mcnt = serve(id0, id0 < MFULL, mcnt, m_full, mlist, cnt_m, False,
                     mrow_hbm, u_win0, e_win0)
        m_fire(id2, 0)
        m_wait(id1, 1)
        mcnt = serve(id1, id1 < MFULL, mcnt, m_full, mlist, cnt_m, False,
                     mrow_hbm, u_win1, e_win1)
        return mcnt

    mcnt = lax.fori_loop(0, MPAIRS, m_pair, 0)

    @pl.when(wid == MFULL % NW)
    def _():
        pltpu.async_copy(mtail_hbm, u_win0.at[:, pl.ds(0, MT_PAD)],
                         sem0).wait()

    mcnt = serve(MFULL, wid == MFULL % NW, mcnt, m_full, mlist, cnt_m,
                 False, mrow_hbm, u_win0, e_win0)
    drain(mcnt)


def _dot_body(blend_hbm, mrow_hbm, out_hbm, b_v, m_v, out_v, sem):
    wid = lax.axis_index("s") * NC + lax.axis_index("c")
    base = wid * BPW
    lanes = lax.iota(jnp.int32, L)
    pltpu.sync_copy(blend_hbm.at[pl.ds(base * K, BPW * K)], b_v)
    pltpu.sync_copy(mrow_hbm.at[pl.ds(base * K, BPW * K)], m_v)

    def group(g, carry):
        acc = jnp.zeros((L,), jnp.float32)
        for j in range(L):
            i = (g * L + j) * K
            blo = b_v[pl.ds(i, L)]
            bhi = b_v[pl.ds(i + L, L)]
            mlo = m_v[pl.ds(i, L)]
            mhi = m_v[pl.ds(i + L, L)]
            s = jnp.sum(blo * mlo + bhi * mhi)
            acc = jnp.where(lanes == j, s, acc)
        out_v[pl.ds(g * L, L)] = acc
        return carry

    lax.fori_loop(0, BPW // L, group, 0)
    pltpu.sync_copy(out_v, out_hbm.at[pl.ds(base, BPW)])


def kernel(o, m, is_user, U, M, E):
    wbit = is_user.reshape(-1).astype(jnp.int32)
    opk = o.astype(jnp.int32) | lax.shift_left(wbit, 30)
    m32 = m.astype(jnp.int32)
    ut = U.T
    mt = M.T
    et = E.T
    utail = jnp.pad(U[UFULL * WIN:].T, ((0, 0), (0, UT_PAD - UT_ROWS)))
    etail = jnp.pad(E[UFULL * WIN:].T, ((0, 0), (0, UT_PAD - UT_ROWS)))
    mtail = jnp.pad(M[MFULL * WIN:].T, ((0, 0), (0, MT_PAD - MT_ROWS)))
    mesh = plsc.VectorSubcoreMesh(core_axis_name="c", subcore_axis_name="s")
    params = pltpu.CompilerParams(needs_layout_passes=False)

    gather = pl.kernel(
        _gather_body,
        mesh=mesh,
        compiler_params=params,
        out_type=(jax.ShapeDtypeStruct((B * K,), jnp.float32),
                  jax.ShapeDtypeStruct((B * K,), jnp.float32)),
        scratch_types=[
            pltpu.VMEM((B,), jnp.int32),
            pltpu.VMEM((B,), jnp.int32),
            pltpu.VMEM((K, WIN), jnp.float32),
            pltpu.VMEM((K, WIN), jnp.float32),
            pltpu.VMEM((K, WIN), jnp.float32),
            pltpu.VMEM((K, WIN), jnp.float32),
            pltpu.VMEM((LCAP,), jnp.int32),
            pltpu.VMEM((LCAP,), jnp.int32),
            pltpu.VMEM((LCAP * K,), jnp.float32),
            pltpu.SemaphoreType.DMA,
            pltpu.SemaphoreType.DMA,
            pltpu.SemaphoreType.DMA,
        ],
    )
    blend, mrow = gather(opk, m32, ut, mt, et, utail, etail, mtail)

    dot = pl.kernel(
        _dot_body,
        mesh=mesh,
        compiler_params=params,
        out_type=jax.ShapeDtypeStruct((B,), jnp.float32),
        scratch_types=[
            pltpu.VMEM((BPW * K,), jnp.float32),
            pltpu.VMEM((BPW * K,), jnp.float32),
            pltpu.VMEM((BPW,), jnp.float32),
            pltpu.SemaphoreType.DMA,
        ],
    )
    return dot(blend, mrow)


# speedup vs baseline: 4.6012x; 1.0842x over previous
import jax
import jax.numpy as jnp
from jax import lax
from jax.experimental import pallas as pl
from jax.experimental.pallas import tpu as pltpu
from jax.experimental.pallas import tpu_sc as plsc

K = 32
L = 16
NC = 2
NS = 16
NW = NC * NS
B = 16384
BPW = B // NW
NU = 1_000_000
NM = 100_000
WIN = 512
WSH = 9
UFULL = NU // WIN
MFULL = NM // WIN
UT_ROWS = NU - UFULL * WIN
MT_ROWS = NM - MFULL * WIN
UT_PAD = 128
MT_PAD = 256
UTRIPS = (UFULL // NW + 3) // 3
MTRIPS = (MFULL // NW + 3) // 3
CHUNK = 2048
LCAP = 768


def _permute(x, idx):
    dnums = lax.GatherDimensionNumbers(
        offset_dims=(), collapsed_slice_dims=(0,), start_index_map=(0,))
    return lax.gather(x, idx[:, None], dnums, (1,),
                      mode=lax.GatherScatterMode.PROMISE_IN_BOUNDS)


def _splat(x):
    return jnp.full((L,), x, jnp.int32)


def _extract(v, l):
    return _permute(v, _splat(l))[0]


def _gather_body(o_hbm, m_hbm, ut_hbm, mt_hbm, et_hbm,
                 utail_hbm, etail_hbm, mtail_hbm,
                 blend_hbm, mrow_hbm,
                 chunk, u_w0, u_w1, u_w2, e_w0, e_w1, e_w2,
                 vul, bul, vml, bml, hrows, sem0, sem1, sem2, sem_h):
    wid = lax.axis_index("s") * NC + lax.axis_index("c")
    lanes = lax.iota(jnp.int32, L)
    k_lo = lanes
    k_hi = lanes + L
    u_bufs = (u_w0, u_w1, u_w2)
    e_bufs = (e_w0, e_w1, e_w2)
    sems = (sem0, sem1, sem2)

    def u_fire(win_id, p):
        @pl.when(win_id < UFULL)
        def _():
            off = win_id * WIN
            pltpu.async_copy(ut_hbm.at[:, pl.ds(off, WIN)], u_bufs[p],
                             sems[p])
            pltpu.async_copy(et_hbm.at[:, pl.ds(off, WIN)], e_bufs[p],
                             sems[p])

    def u_wait(win_id, p):
        @pl.when(win_id < UFULL)
        def _():
            pltpu.make_async_copy(
                ut_hbm.at[:, pl.ds(0, WIN)], u_bufs[p], sems[p]).wait()
            pltpu.make_async_copy(
                et_hbm.at[:, pl.ds(0, WIN)], e_bufs[p], sems[p]).wait()

    u_fire(wid, 0)
    u_fire(wid + NW, 1)

    def make_scan(src_hbm, vdst, bdst):
        def outer(ci, cnt):
            pltpu.sync_copy(src_hbm.at[pl.ds(ci * CHUNK, CHUNK)], chunk)

            def scan(g, cnt):
                v = chunk[pl.ds(g * L, L)]
                own = jnp.bitwise_and(lax.shift_right_logical(v, WSH), NW - 1)
                msk = own == wid
                pos = cnt + plsc.cumsum(msk.astype(jnp.int32)) - 1
                pos = jnp.minimum(pos, LCAP - 1)
                plsc.store_scatter(vdst, [pos], v, mask=msk)
                plsc.store_scatter(bdst, [pos], ci * CHUNK + g * L + lanes,
                                   mask=msk)
                return cnt + plsc.all_reduce_population_count(msk)[0]

            return lax.fori_loop(0, CHUNK // L, scan, cnt)
        return lax.fori_loop(0, B // CHUNK, outer, 0)

    cnt_u = jnp.minimum(make_scan(o_hbm, vul, bul), LCAP)
    cnt_m = jnp.minimum(make_scan(m_hbm, vml, bml), LCAP)

    def serve(win_id, win_ok, hcnt, vlist, blist, src_cnt, blended,
              dst_hbm, u_ref, e_ref):
        ngrp = lax.shift_right_logical(src_cnt + L - 1, 4)

        def grp(i, hc):
            vvec = vlist[pl.ds(i * L, L)]
            bvec = jnp.bitwise_and(blist[pl.ds(i * L, L)], B - 1)
            win_of = jnp.bitwise_and(
                lax.shift_right_logical(vvec, WSH), 2047)
            valid = (i * L + lanes) < src_cnt
            msk = (win_of == win_id) & valid & win_ok

            def hit(h, st):
                c, mc = st
                l = plsc.all_reduce_ffs(mc)[0]
                b = _extract(bvec, l)
                v = _extract(vvec, l)
                colv = _splat(jnp.bitwise_and(v, WIN - 1))
                a_lo = plsc.load_gather(u_ref, [k_lo, colv])
                a_hi = plsc.load_gather(u_ref, [k_hi, colv])
                if blended:
                    e_lo = plsc.load_gather(e_ref, [k_lo, colv])
                    e_hi = plsc.load_gather(e_ref, [k_hi, colv])
                    wf = jnp.float32(
                        jnp.bitwise_and(lax.shift_right_logical(v, 30), 1))
                    wb = jnp.full((L,), wf, jnp.float32)
                    r_lo = a_lo * wb + e_lo * (1.0 - wb)
                    r_hi = a_hi * wb + e_hi * (1.0 - wb)
                else:
                    r_lo, r_hi = a_lo, a_hi
                slot = jnp.minimum(c, LCAP - 1) * K
                hrows[pl.ds(slot, L)] = r_lo
                hrows[pl.ds(slot + L, L)] = r_hi
                pltpu.async_copy(hrows.at[pl.ds(slot, K)],
                                 dst_hbm.at[pl.ds(b * K, K)], sem_h)
                return c + 1, mc & (lanes != l)

            pc = plsc.all_reduce_population_count(msk)[0]
            hc, _ = lax.fori_loop(0, pc, hit, (hc, msk))
            return hc

        return lax.fori_loop(0, ngrp, grp, hcnt)

    def drain(n):
        def one(i, c):
            pltpu.make_async_copy(
                blend_hbm.at[pl.ds(0, K)], hrows.at[pl.ds(0, K)], sem_h).wait()
            return c
        lax.fori_loop(0, n, one, 0)

    def u_trip(t, hcnt):
        for j in range(3):
            idj = wid + (3 * t + j) * NW
            u_fire(wid + (3 * t + j + 2) * NW, (j + 2) % 3)
            u_wait(idj, j)
            hcnt = serve(idj, idj < UFULL, hcnt, vul, bul, cnt_u, True,
                         blend_hbm, u_bufs[j], e_bufs[j])
        return hcnt

    hcnt = lax.fori_loop(0, UTRIPS, u_trip, 0)

    @pl.when(wid == UFULL % NW)
    def _():
        c1 = pltpu.async_copy(utail_hbm, u_w0.at[:, pl.ds(0, UT_PAD)], sem0)
        c2 = pltpu.async_copy(etail_hbm, e_w0.at[:, pl.ds(0, UT_PAD)], sem0)
        c1.wait()
        c2.wait()

    hcnt = serve(UFULL, wid == UFULL % NW, hcnt, vul, bul, cnt_u,
                 True, blend_hbm, u_w0, e_w0)
    drain(hcnt)

    def m_fire(win_id, p):
        @pl.when(win_id < MFULL)
        def _():
            pltpu.async_copy(mt_hbm.at[:, pl.ds(win_id * WIN, WIN)],
                             u_bufs[p], sems[p])

    def m_wait(win_id, p):
        @pl.when(win_id < MFULL)
        def _():
            pltpu.make_async_copy(
                mt_hbm.at[:, pl.ds(0, WIN)], u_bufs[p], sems[p]).wait()

    m_fire(wid, 0)
    m_fire(wid + NW, 1)

    def m_trip(t, mcnt):
        for j in range(3):
            idj = wid + (3 * t + j) * NW
            m_fire(wid + (3 * t + j + 2) * NW, (j + 2) % 3)
            m_wait(idj, j)
            mcnt = serve(idj, idj < MFULL, mcnt, vml, bml, cnt_m, False,
                         mrow_hbm, u_bufs[j], e_bufs[j])
        return mcnt

    mcnt = lax.fori_loop(0, MTRIPS, m_trip, 0)

    @pl.when(wid == MFULL % NW)
    def _():
        pltpu.async_copy(mtail_hbm, u_w0.at[:, pl.ds(0, MT_PAD)],
                         sem0).wait()

    mcnt = serve(MFULL, wid == MFULL % NW, mcnt, vml, bml, cnt_m,
                 False, mrow_hbm, u_w0, e_w0)
    drain(mcnt)


def _dot_body(blend_hbm, mrow_hbm, out_hbm, b_v, m_v, out_v, sem):
    wid = lax.axis_index("s") * NC + lax.axis_index("c")
    base = wid * BPW
    lanes = lax.iota(jnp.int32, L)
    pltpu.sync_copy(blend_hbm.at[pl.ds(base * K, BPW * K)], b_v)
    pltpu.sync_copy(mrow_hbm.at[pl.ds(base * K, BPW * K)], m_v)

    def group(g, carry):
        acc = jnp.zeros((L,), jnp.float32)
        for j in range(L):
            i = (g * L + j) * K
            blo = b_v[pl.ds(i, L)]
            bhi = b_v[pl.ds(i + L, L)]
            mlo = m_v[pl.ds(i, L)]
            mhi = m_v[pl.ds(i + L, L)]
            s = jnp.sum(blo * mlo + bhi * mhi)
            acc = jnp.where(lanes == j, s, acc)
        out_v[pl.ds(g * L, L)] = acc
        return carry

    lax.fori_loop(0, BPW // L, group, 0)
    pltpu.sync_copy(out_v, out_hbm.at[pl.ds(base, BPW)])


def kernel(o, m, is_user, U, M, E):
    wbit = is_user.reshape(-1).astype(jnp.int32)
    opk = o.astype(jnp.int32) | lax.shift_left(wbit, 30)
    m32 = m.astype(jnp.int32)
    ut = U.T
    mt = M.T
    et = E.T
    utail = jnp.pad(U[UFULL * WIN:].T, ((0, 0), (0, UT_PAD - UT_ROWS)))
    etail = jnp.pad(E[UFULL * WIN:].T, ((0, 0), (0, UT_PAD - UT_ROWS)))
    mtail = jnp.pad(M[MFULL * WIN:].T, ((0, 0), (0, MT_PAD - MT_ROWS)))
    mesh = plsc.VectorSubcoreMesh(core_axis_name="c", subcore_axis_name="s")
    params = pltpu.CompilerParams(needs_layout_passes=False)

    gather = pl.kernel(
        _gather_body,
        mesh=mesh,
        compiler_params=params,
        out_type=(jax.ShapeDtypeStruct((B * K,), jnp.float32),
                  jax.ShapeDtypeStruct((B * K,), jnp.float32)),
        scratch_types=[
            pltpu.VMEM((CHUNK,), jnp.int32),
            pltpu.VMEM((K, WIN), jnp.float32),
            pltpu.VMEM((K, WIN), jnp.float32),
            pltpu.VMEM((K, WIN), jnp.float32),
            pltpu.VMEM((K, WIN), jnp.float32),
            pltpu.VMEM((K, WIN), jnp.float32),
            pltpu.VMEM((K, WIN), jnp.float32),
            pltpu.VMEM((LCAP,), jnp.int32),
            pltpu.VMEM((LCAP,), jnp.int32),
            pltpu.VMEM((LCAP,), jnp.int32),
            pltpu.VMEM((LCAP,), jnp.int32),
            pltpu.VMEM((LCAP * K,), jnp.float32),
            pltpu.SemaphoreType.DMA,
            pltpu.SemaphoreType.DMA,
            pltpu.SemaphoreType.DMA,
            pltpu.SemaphoreType.DMA,
        ],
    )
    blend, mrow = gather(opk, m32, ut, mt, et, utail, etail, mtail)

    dot = pl.kernel(
        _dot_body,
        mesh=mesh,
        compiler_params=params,
        out_type=jax.ShapeDtypeStruct((B,), jnp.float32),
        scratch_types=[
            pltpu.VMEM((BPW * K,), jnp.float32),
            pltpu.VMEM((BPW * K,), jnp.float32),
            pltpu.VMEM((BPW,), jnp.float32),
            pltpu.SemaphoreType.DMA,
        ],
    )
    return dot(blend, mrow)
